# baseline (speedup 1.0000x reference)
import functools

import jax
import jax.numpy as jnp
from jax import lax
from jax.experimental import pallas as pl
from jax.experimental.pallas import tpu as pltpu
from jax.experimental.pallas import tpu_sc as plsc

NC, NS = 2, 16
NW = NC * NS
CH = 128


def _sc_gather_one_t(idx, table_t):
    B = idx.shape[0]
    D = table_t.shape[0]
    b_per_w = B // NW
    n_ch = b_per_w // CH

    idx3 = idx.reshape(NW, n_ch, CH).astype(jnp.int32)

    mesh = plsc.VectorSubcoreMesh(core_axis_name="c", subcore_axis_name="s")

    @functools.partial(
        pl.kernel,
        out_type=jax.ShapeDtypeStruct((D, B), jnp.float32),
        mesh=mesh,
        scratch_types=[
            pltpu.VMEM((n_ch, CH), jnp.int32),
            pltpu.VMEM((D, b_per_w), jnp.float32),
            pltpu.SemaphoreType.DMA,
        ],
        compiler_params=pltpu.CompilerParams(use_tc_tiling_on_sc=False),
    )
    def gather_kernel(idx_hbm, tab_hbm, out_hbm, idx_v, rows_v, sem):
        wid = lax.axis_index("s") * NC + lax.axis_index("c")
        base = wid * b_per_w
        pltpu.sync_copy(idx_hbm.at[wid], idx_v)
        for cp in range(D):
            copies = []
            for j in range(n_ch):
                copies.append(pltpu.async_copy(
                    tab_hbm.at[cp].at[idx_v.at[j]],
                    rows_v.at[cp].at[pl.ds(j * CH, CH)], sem))
            for c in copies:
                c.wait()
        pltpu.sync_copy(rows_v, out_hbm.at[slice(None), pl.ds(base, b_per_w)])

    return gather_kernel(idx3, table_t)


def _mlp_body(u_ref, o_ref, w1u_ref, w1o_ref, b1_ref, w2_ref, b2_ref,
              w3t_ref, b3_ref, out_ref):
    h1 = jnp.dot(u_ref[...], w1u_ref[...], preferred_element_type=jnp.float32)
    h1 = h1 + jnp.dot(o_ref[...], w1o_ref[...],
                      preferred_element_type=jnp.float32)
    h1 = jnp.maximum(h1 + b1_ref[...], 0.0)
    h2 = jnp.dot(h1, w2_ref[...], preferred_element_type=jnp.float32)
    h2 = jnp.maximum(h2 + b2_ref[...], 0.0)
    out_ref[...] = (jnp.sum(h2 * w3t_ref[...], axis=1, keepdims=True)
                    + b3_ref[...])


def _mlp(u_emb, o_emb, W1, b1, W2, b2, W3, b3, chunk=2048):
    B, D = u_emb.shape
    H1 = W1.shape[1]
    H2 = W2.shape[1]
    w1u = W1[:D]
    w1o = W1[D:]
    b1r = b1.reshape(1, H1)
    b2r = b2.reshape(1, H2)
    w3t = W3.reshape(1, H2)
    b3r = b3.reshape(1, 1)
    grid = (B // chunk,)
    return pl.pallas_call(
        _mlp_body,
        grid=grid,
        in_specs=[
            pl.BlockSpec((chunk, D), lambda i: (i, 0)),
            pl.BlockSpec((chunk, D), lambda i: (i, 0)),
            pl.BlockSpec((D, H1), lambda i: (0, 0)),
            pl.BlockSpec((D, H1), lambda i: (0, 0)),
            pl.BlockSpec((1, H1), lambda i: (0, 0)),
            pl.BlockSpec((H1, H2), lambda i: (0, 0)),
            pl.BlockSpec((1, H2), lambda i: (0, 0)),
            pl.BlockSpec((1, H2), lambda i: (0, 0)),
            pl.BlockSpec((1, 1), lambda i: (0, 0)),
        ],
        out_specs=pl.BlockSpec((chunk, 1), lambda i: (i, 0)),
        out_shape=jax.ShapeDtypeStruct((B, 1), jnp.float32),
    )(u_emb, o_emb, w1u, w1o, b1r, W2, b2r, w3t, b3r)


def kernel(user_id, destination, user_table, origin_table,
           W1, b1, W2, b2, W3, b3):
    vp = user_table.shape[0] + (-user_table.shape[0]) % 128
    pad_n = vp - user_table.shape[0]
    ut_p = jnp.pad(user_table.T, ((0, 0), (0, pad_n)))
    ot_p = jnp.pad(origin_table.T, ((0, 0), (0, pad_n)))
    u_emb = _sc_gather_one_t(user_id, ut_p).T
    o_emb = _sc_gather_one_t(destination, ot_p).T
    return _mlp(u_emb, o_emb, W1, b1, W2, b2, W3, b3)


# speedup vs baseline: 1.1796x; 1.1796x over previous
import functools

import jax
import jax.numpy as jnp
from jax import lax
from jax.experimental import pallas as pl
from jax.experimental.pallas import tpu as pltpu
from jax.experimental.pallas import tpu_sc as plsc

NC, NS = 2, 16
NW = NC * NS
CH = 128


def _sc_gather_one(idx, table):
    B = idx.shape[0]
    D = table.shape[1]
    b_per_w = B // NW
    n_ch = b_per_w // CH

    idx3 = idx.reshape(NW, n_ch, CH).astype(jnp.int32)

    mesh = plsc.VectorSubcoreMesh(core_axis_name="c", subcore_axis_name="s")

    @functools.partial(
        pl.kernel,
        out_type=jax.ShapeDtypeStruct((B, D), jnp.float32),
        mesh=mesh,
        scratch_types=[
            pltpu.VMEM((n_ch, CH), jnp.int32),
            pltpu.VMEM((b_per_w, D), jnp.float32),
            pltpu.SemaphoreType.DMA,
        ],
        compiler_params=pltpu.CompilerParams(use_tc_tiling_on_sc=False),
    )
    def gather_kernel(idx_hbm, tab_hbm, out_hbm, idx_v, rows_v, sem):
        wid = lax.axis_index("s") * NC + lax.axis_index("c")
        base = wid * b_per_w
        pltpu.sync_copy(idx_hbm.at[wid], idx_v)
        copies = []
        for j in range(n_ch):
            copies.append(pltpu.async_copy(
                tab_hbm.at[idx_v.at[j]], rows_v.at[pl.ds(j * CH, CH)], sem))
        for c in copies:
            c.wait()
        pltpu.sync_copy(rows_v, out_hbm.at[pl.ds(base, b_per_w)])

    return gather_kernel(idx3, table)


def _mlp_body(u_ref, o_ref, w1u_ref, w1o_ref, b1_ref, w2_ref, b2_ref,
              w3t_ref, b3_ref, out_ref):
    h1 = jnp.dot(u_ref[...], w1u_ref[...], preferred_element_type=jnp.float32)
    h1 = h1 + jnp.dot(o_ref[...], w1o_ref[...],
                      preferred_element_type=jnp.float32)
    h1 = jnp.maximum(h1 + b1_ref[...], 0.0)
    h2 = jnp.dot(h1, w2_ref[...], preferred_element_type=jnp.float32)
    h2 = jnp.maximum(h2 + b2_ref[...], 0.0)
    out_ref[...] = (jnp.sum(h2 * w3t_ref[...], axis=1, keepdims=True)
                    + b3_ref[...])


def _mlp(u_emb, o_emb, W1, b1, W2, b2, W3, b3, chunk=2048):
    B, D = u_emb.shape
    H1 = W1.shape[1]
    H2 = W2.shape[1]
    w1u = W1[:D]
    w1o = W1[D:]
    b1r = b1.reshape(1, H1)
    b2r = b2.reshape(1, H2)
    w3t = W3.reshape(1, H2)
    b3r = b3.reshape(1, 1)
    grid = (B // chunk,)
    return pl.pallas_call(
        _mlp_body,
        grid=grid,
        in_specs=[
            pl.BlockSpec((chunk, D), lambda i: (i, 0)),
            pl.BlockSpec((chunk, D), lambda i: (i, 0)),
            pl.BlockSpec((D, H1), lambda i: (0, 0)),
            pl.BlockSpec((D, H1), lambda i: (0, 0)),
            pl.BlockSpec((1, H1), lambda i: (0, 0)),
            pl.BlockSpec((H1, H2), lambda i: (0, 0)),
            pl.BlockSpec((1, H2), lambda i: (0, 0)),
            pl.BlockSpec((1, H2), lambda i: (0, 0)),
            pl.BlockSpec((1, 1), lambda i: (0, 0)),
        ],
        out_specs=pl.BlockSpec((chunk, 1), lambda i: (i, 0)),
        out_shape=jax.ShapeDtypeStruct((B, 1), jnp.float32),
    )(u_emb, o_emb, w1u, w1o, b1r, W2, b2r, w3t, b3r)


def kernel(user_id, destination, user_table, origin_table,
           W1, b1, W2, b2, W3, b3):
    u_emb = _sc_gather_one(user_id, user_table)
    o_emb = _sc_gather_one(destination, origin_table)
    return _mlp(u_emb, o_emb, W1, b1, W2, b2, W3, b3)
